# baseline (speedup 1.0000x reference)
import functools

import jax
import jax.numpy as jnp
from jax import lax
from jax.experimental import pallas as pl
from jax.experimental.pallas import tpu as pltpu
from jax.experimental.pallas import tpu_sc as plsc

NC = 2
NS = 16
NW = NC * NS
NG_EDGE = 2
NR = NW // NG_EDGE
CHUNK = 2000
NGRP = CHUNK // 16
NWIN = 8
NF = 8
SG = 64

_GDN = lax.GatherDimensionNumbers(
    offset_dims=(), collapsed_slice_dims=(0,), start_index_map=(0,)
)


def _vgather(v, idx):
    return lax.gather(
        v, idx[:, None], _GDN, (1,),
        mode=lax.GatherScatterMode.PROMISE_IN_BOUNDS,
    )


def _sc_segment_max(x, src, dst, rows_per_r):
    n, d = x.shape
    e = src.shape[0]
    eh = e // NG_EDGE
    nchunk = eh // CHUNK
    out_rows = NR * rows_per_r

    mesh = plsc.VectorSubcoreMesh(
        core_axis_name="c", subcore_axis_name="s", num_cores=NC, num_subcores=NS
    )

    @functools.partial(
        pl.kernel,
        out_type=jax.ShapeDtypeStruct((NF, NG_EDGE * out_rows * 16), jnp.float32),
        mesh=mesh,
        scratch_types=[
            pltpu.VMEM((2 * (CHUNK + 48),), jnp.int32),
            pltpu.VMEM((2 * (CHUNK + 48),), jnp.int32),
            pltpu.VMEM((16 * NWIN,), jnp.int32),
            pltpu.VMEM((16 * NWIN,), jnp.int32),
            pltpu.VMEM((CHUNK + 128,), jnp.int32),
            pltpu.VMEM((CHUNK + 128,), jnp.int32),
            pltpu.VMEM((2, SG, d), jnp.float32),
        ] + [
            pltpu.VMEM(((rows_per_r + 1) * 16,), jnp.float32) for _ in range(NF)
        ] + [
            pltpu.SemaphoreType.DMA,
            pltpu.SemaphoreType.DMA,
            pltpu.SemaphoreType.DMA,
            pltpu.SemaphoreType.DMA,
        ],
        compiler_params=pltpu.CompilerParams(needs_layout_passes=False),
    )
    def k(x_hbm, src_hbm, dst_hbm, out_hbm, dstc, srcc, cntb, offb,
          idxb, ldb, rows, *rest):
        aggs = rest[:NF]
        sem0, sem1, semd, sems = rest[NF], rest[NF + 1], rest[NF + 2], rest[NF + 3]
        c = lax.axis_index("c")
        s = lax.axis_index("s")
        wid = s * NC + c
        g_half = wid % NG_EDGE
        r = wid // NG_EDGE
        lo = r * rows_per_r
        lane = lax.iota(jnp.int32, 16)

        neg = jnp.full((16,), -jnp.inf, jnp.float32)
        zeros = jnp.zeros((16,), jnp.int32)
        trash = jnp.full((16,), rows_per_r, jnp.int32)
        big = jnp.full((16,), out_rows, jnp.int32)
        lo_v = jnp.full((16,), lo, jnp.int32)
        hi_v = jnp.full((16,), lo + rows_per_r, jnp.int32)

        def init_row(rr, carry):
            for f in range(NF):
                aggs[f][pl.ds(rr * 16, 16)] = neg
            return carry

        lax.fori_loop(0, rows_per_r + 1, init_row, 0)

        def init_buf(i, carry):
            idxb[pl.ds(i * 16, 16)] = zeros
            ldb[pl.ds(i * 16, 16)] = trash
            return carry

        lax.fori_loop(0, (CHUNK + 128) // 16, init_buf, 0)

        for bb in range(2):
            for i in range(3):
                dstc[pl.ds(bb * (CHUNK + 48) + CHUNK + i * 16, 16)] = big
                srcc[pl.ds(bb * (CHUNK + 48) + CHUNK + i * 16, 16)] = zeros

        ebase = g_half * eh

        def fire_chunk(ci, bb):
            base = ebase + ci * CHUNK
            pltpu.async_copy(
                dst_hbm.at[pl.ds(base, CHUNK)],
                dstc.at[pl.ds(bb * (CHUNK + 48), CHUNK)], semd,
            )
            pltpu.async_copy(
                src_hbm.at[pl.ds(base, CHUNK)],
                srcc.at[pl.ds(bb * (CHUNK + 48), CHUNK)], sems,
            )

        def drain_chunk(bb):
            pltpu.make_async_copy(
                dst_hbm.at[pl.ds(0, CHUNK)],
                dstc.at[pl.ds(bb * (CHUNK + 48), CHUNK)], semd,
            ).wait()
            pltpu.make_async_copy(
                src_hbm.at[pl.ds(0, CHUNK)],
                srcc.at[pl.ds(bb * (CHUNK + 48), CHUNK)], sems,
            ).wait()

        fire_chunk(0, 0)

        def chunk_body(ci, carry):
            cb = ci % 2
            cbase = cb * (CHUNK + 48)
            drain_chunk(cb)

            @pl.when((cb == 0) & (ci + 1 < nchunk))
            def _():
                fire_chunk(ci + 1, 1)

            @pl.when((cb == 1) & (ci + 1 < nchunk))
            def _():
                fire_chunk(ci + 1, 0)

            def cnt_win(w, carry):
                b = w * 256
                acc = zeros
                for u in range(16):
                    dv = dstc[pl.ds(cbase + b + u * 16, 16)]
                    m = (dv >= lo_v) & (dv < hi_v)
                    cv = plsc.all_reduce_population_count(m)
                    acc = jnp.where(lane == u, cv, acc)
                cntb[pl.ds(w * 16, 16)] = acc
                return carry

            lax.fori_loop(0, NWIN, cnt_win, 0)

            off = jnp.zeros((16,), jnp.int32)
            l15 = jnp.full((16,), 15, jnp.int32)
            for w in range(NWIN):
                cv = cntb[pl.ds(w * 16, 16)]
                cs = jnp.cumsum(cv)
                offb[pl.ds(w * 16, 16)] = off + (cs - cv)
                off = off + _vgather(cs, l15)
            cnt = jnp.max(off)

            def scat_win(w, carry):
                b = w * 256
                ov = offb[pl.ds(w * 16, 16)]
                for u in range(16):
                    dv = dstc[pl.ds(cbase + b + u * 16, 16)]
                    sv = srcc[pl.ds(cbase + b + u * 16, 16)]
                    m = (dv >= lo_v) & (dv < hi_v)
                    mi = m.astype(jnp.int32)
                    cs = jnp.cumsum(mi)
                    osp = _vgather(ov, jnp.full((16,), u, jnp.int32))
                    addr = cs + osp - 1
                    plsc.store_scatter(idxb, [addr], sv, mask=m)
                    plsc.store_scatter(ldb, [addr], dv - lo_v, mask=m)
                return carry

            lax.fori_loop(0, NWIN, scat_win, 0)

            nsg2 = (cnt + 2 * SG - 1) // (2 * SG)

            def fire(sg, buf, sem):
                pltpu.async_copy(
                    x_hbm.at[idxb.at[pl.ds(sg * SG, SG)]], rows.at[buf], sem
                )

            def drain(buf, sem):
                pltpu.make_async_copy(
                    x_hbm.at[idxb.at[pl.ds(0, SG)]], rows.at[buf], sem
                ).wait()

            def fold(sg, buf):
                for q in range(SG // 16):
                    ldv = ldb[pl.ds(sg * SG + q * 16, 16)]
                    base = ldv * 16
                    addrs = [
                        _vgather(base, jnp.full((16,), j, jnp.int32)) + lane
                        for j in range(16)
                    ]
                    for j in range(16):
                        for f in range(NF):
                            cur = plsc.load_gather(aggs[f], [addrs[j]])
                            xv = rows[buf, q * 16 + j, pl.ds(f * 16, 16)]
                            plsc.store_scatter(
                                aggs[f], [addrs[j]], jnp.maximum(cur, xv)
                            )

            @pl.when(nsg2 > 0)
            def _():
                fire(0, 0, sem0)

            def apply_pair(gi, carry):
                fire(2 * gi + 1, 1, sem1)
                drain(0, sem0)
                fold(2 * gi, 0)

                @pl.when(gi + 1 < nsg2)
                def _():
                    fire(2 * gi + 2, 0, sem0)

                drain(1, sem1)
                fold(2 * gi + 1, 1)
                return carry

            lax.fori_loop(0, nsg2, apply_pair, 0)
            return carry

        lax.fori_loop(0, nchunk, chunk_body, 0)

        obase = (g_half * out_rows + lo) * 16
        for f in range(NF):
            pltpu.sync_copy(
                aggs[f].at[pl.ds(0, rows_per_r * 16)],
                out_hbm.at[f, pl.ds(obase, rows_per_r * 16)],
            )

    return k(x, src, dst)


def _tc_body(*refs):
    a0s = refs[:NF]
    a1s = refs[NF:2 * NF]
    x_ref, wl_ref, wr_ref, b_ref, o_ref = refs[2 * NF:]
    a = jnp.maximum(
        jnp.concatenate([ref[...] for ref in a0s], axis=1),
        jnp.concatenate([ref[...] for ref in a1s], axis=1),
    )
    a = jnp.where(jnp.isfinite(a), a, 0.0)
    y = jnp.dot(a, wl_ref[...], preferred_element_type=jnp.float32)
    y = y + jnp.dot(x_ref[...], wr_ref[...], preferred_element_type=jnp.float32)
    y = y + b_ref[...]
    m = jnp.max(y, axis=1, keepdims=True)
    z = y - m
    lse = jnp.log(jnp.sum(jnp.exp(z), axis=1, keepdims=True))
    o_ref[...] = z - lse


def _tc_final(a0s, a1s, x, w_l, w_r, b2):
    n, d = x.shape
    c = w_l.shape[1]
    blk = 1000
    grid = n // blk
    bank_spec = pl.BlockSpec((blk, 16), lambda i: (i, 0))
    return pl.pallas_call(
        _tc_body,
        grid=(grid,),
        in_specs=(
            [bank_spec] * (2 * NF)
            + [
                pl.BlockSpec((blk, d), lambda i: (i, 0)),
                pl.BlockSpec((d, c), lambda i: (0, 0)),
                pl.BlockSpec((d, c), lambda i: (0, 0)),
                pl.BlockSpec((1, c), lambda i: (0, 0)),
            ]
        ),
        out_specs=pl.BlockSpec((blk, c), lambda i: (i, 0)),
        out_shape=jax.ShapeDtypeStruct((n, c), jnp.float32),
    )(*a0s, *a1s, x, w_l, w_r, b2)


def kernel(x, edge_index, W_l, W_r, b):
    n, d = x.shape
    e = edge_index.shape[1]
    c = W_l.shape[1]
    rows_per_r = -(-n // (NR * 8)) * 8
    out_rows = NR * rows_per_r

    src = edge_index[0]
    dst = edge_index[1]
    pad = (-e) % (NG_EDGE * CHUNK)
    if pad:
        src = jnp.concatenate([src, jnp.zeros((pad,), jnp.int32)])
        dst = jnp.concatenate([dst, jnp.full((pad,), out_rows, jnp.int32)])

    parts = _sc_segment_max(x, src, dst, rows_per_r)
    banks = parts.reshape(NF, NG_EDGE * out_rows, 16)
    a0s = [banks[f, :n] for f in range(NF)]
    a1s = [banks[f, out_rows:out_rows + n] for f in range(NF)]
    return _tc_final(a0s, a1s, x, W_l, W_r, b.reshape(1, c))


# speedup vs baseline: 1.0026x; 1.0026x over previous
import functools

import jax
import jax.numpy as jnp
from jax import lax
from jax.experimental import pallas as pl
from jax.experimental.pallas import tpu as pltpu
from jax.experimental.pallas import tpu_sc as plsc

NC = 2
NS = 16
NW = NC * NS
NG_EDGE = 2
NR = NW // NG_EDGE
CHUNK = 2000
NGRP = CHUNK // 16
NWIN = 8
NF = 8
SG = 64

_GDN = lax.GatherDimensionNumbers(
    offset_dims=(), collapsed_slice_dims=(0,), start_index_map=(0,)
)


def _vgather(v, idx):
    return lax.gather(
        v, idx[:, None], _GDN, (1,),
        mode=lax.GatherScatterMode.PROMISE_IN_BOUNDS,
    )


def _sc_segment_max(x, src, dst, rows_per_r):
    n, d = x.shape
    e = src.shape[0]
    eh = e // NG_EDGE
    nchunk = eh // CHUNK
    out_rows = NR * rows_per_r

    mesh = plsc.VectorSubcoreMesh(
        core_axis_name="c", subcore_axis_name="s", num_cores=NC, num_subcores=NS
    )

    @functools.partial(
        pl.kernel,
        out_type=jax.ShapeDtypeStruct((NF, NG_EDGE * out_rows * 16), jnp.float32),
        mesh=mesh,
        scratch_types=[
            pltpu.VMEM((2 * (CHUNK + 48),), jnp.int32),
            pltpu.VMEM((2 * (CHUNK + 48),), jnp.int32),
            pltpu.VMEM((16 * NWIN,), jnp.int32),
            pltpu.VMEM((16 * NWIN,), jnp.int32),
            pltpu.VMEM((CHUNK + 128,), jnp.int32),
            pltpu.VMEM((CHUNK + 128,), jnp.int32),
            pltpu.VMEM((2, SG, d), jnp.float32),
        ] + [
            pltpu.VMEM(((rows_per_r + 1) * 16,), jnp.float32) for _ in range(NF)
        ] + [
            pltpu.SemaphoreType.DMA,
            pltpu.SemaphoreType.DMA,
            pltpu.SemaphoreType.DMA,
            pltpu.SemaphoreType.DMA,
        ],
        compiler_params=pltpu.CompilerParams(needs_layout_passes=False),
    )
    def k(x_hbm, src_hbm, dst_hbm, out_hbm, dstc, srcc, cntb, offb,
          idxb, ldb, rows, *rest):
        aggs = rest[:NF]
        sem0, sem1, semd, sems = rest[NF], rest[NF + 1], rest[NF + 2], rest[NF + 3]
        c = lax.axis_index("c")
        s = lax.axis_index("s")
        wid = s * NC + c
        g_half = wid % NG_EDGE
        r = wid // NG_EDGE
        lo = r * rows_per_r
        lane = lax.iota(jnp.int32, 16)

        neg = jnp.full((16,), -jnp.inf, jnp.float32)
        zeros = jnp.zeros((16,), jnp.int32)
        trash = jnp.full((16,), rows_per_r, jnp.int32)
        big = jnp.full((16,), out_rows, jnp.int32)
        lo_v = jnp.full((16,), lo, jnp.int32)
        hi_v = jnp.full((16,), lo + rows_per_r, jnp.int32)

        def init_row(rr, carry):
            for f in range(NF):
                aggs[f][pl.ds(rr * 16, 16)] = neg
            return carry

        lax.fori_loop(0, rows_per_r + 1, init_row, 0)

        def init_buf(i, carry):
            idxb[pl.ds(i * 16, 16)] = zeros
            ldb[pl.ds(i * 16, 16)] = trash
            return carry

        lax.fori_loop(0, (CHUNK + 128) // 16, init_buf, 0)

        for bb in range(2):
            for i in range(3):
                dstc[pl.ds(bb * (CHUNK + 48) + CHUNK + i * 16, 16)] = big
                srcc[pl.ds(bb * (CHUNK + 48) + CHUNK + i * 16, 16)] = zeros

        ebase = g_half * eh

        def fire_chunk(ci, bb):
            base = ebase + ci * CHUNK
            pltpu.async_copy(
                dst_hbm.at[pl.ds(base, CHUNK)],
                dstc.at[pl.ds(bb * (CHUNK + 48), CHUNK)], semd,
            )
            pltpu.async_copy(
                src_hbm.at[pl.ds(base, CHUNK)],
                srcc.at[pl.ds(bb * (CHUNK + 48), CHUNK)], sems,
            )

        def drain_chunk(bb):
            pltpu.make_async_copy(
                dst_hbm.at[pl.ds(0, CHUNK)],
                dstc.at[pl.ds(bb * (CHUNK + 48), CHUNK)], semd,
            ).wait()
            pltpu.make_async_copy(
                src_hbm.at[pl.ds(0, CHUNK)],
                srcc.at[pl.ds(bb * (CHUNK + 48), CHUNK)], sems,
            ).wait()

        fire_chunk(0, 0)

        def chunk_body(ci, carry):
            cb = ci % 2
            cbase = cb * (CHUNK + 48)
            drain_chunk(cb)

            @pl.when((cb == 0) & (ci + 1 < nchunk))
            def _():
                fire_chunk(ci + 1, 1)

            @pl.when((cb == 1) & (ci + 1 < nchunk))
            def _():
                fire_chunk(ci + 1, 0)

            def cnt_win(w, carry):
                b = w * 256
                acc = zeros
                for u in range(16):
                    dv = dstc[pl.ds(cbase + b + u * 16, 16)]
                    m = (dv >= lo_v) & (dv < hi_v)
                    cv = plsc.all_reduce_population_count(m)
                    acc = jnp.where(lane == u, cv, acc)
                cntb[pl.ds(w * 16, 16)] = acc
                return carry

            lax.fori_loop(0, NWIN, cnt_win, 0)

            off = jnp.zeros((16,), jnp.int32)
            l15 = jnp.full((16,), 15, jnp.int32)
            for w in range(NWIN):
                cv = cntb[pl.ds(w * 16, 16)]
                cs = jnp.cumsum(cv)
                offb[pl.ds(w * 16, 16)] = off + (cs - cv)
                off = off + _vgather(cs, l15)
            cnt = jnp.max(off)

            def scat_win(w, carry):
                b = w * 256
                ov = offb[pl.ds(w * 16, 16)]
                for u in range(16):
                    dv = dstc[pl.ds(cbase + b + u * 16, 16)]
                    sv = srcc[pl.ds(cbase + b + u * 16, 16)]
                    m = (dv >= lo_v) & (dv < hi_v)
                    mi = m.astype(jnp.int32)
                    cs = jnp.cumsum(mi)
                    osp = _vgather(ov, jnp.full((16,), u, jnp.int32))
                    addr = cs + osp - 1
                    plsc.store_scatter(idxb, [addr], sv, mask=m)
                    plsc.store_scatter(ldb, [addr], dv - lo_v, mask=m)
                return carry

            lax.fori_loop(0, NWIN, scat_win, 0)

            nsg2 = (cnt + 2 * SG - 1) // (2 * SG)

            def fire(sg, buf, sem):
                pltpu.async_copy(
                    x_hbm.at[idxb.at[pl.ds(sg * SG, SG)]], rows.at[buf], sem
                )

            def drain(buf, sem):
                pltpu.make_async_copy(
                    x_hbm.at[idxb.at[pl.ds(0, SG)]], rows.at[buf], sem
                ).wait()

            def fold(sg, buf):
                def foldq(q, carry):
                    ldv = ldb[pl.ds(sg * SG + q * 16, 16)]
                    base = ldv * 16
                    addrs = [
                        _vgather(base, jnp.full((16,), j, jnp.int32)) + lane
                        for j in range(16)
                    ]
                    for j in range(16):
                        for f in range(NF):
                            cur = plsc.load_gather(aggs[f], [addrs[j]])
                            xv = rows[buf, q * 16 + j, pl.ds(f * 16, 16)]
                            plsc.store_scatter(
                                aggs[f], [addrs[j]], jnp.maximum(cur, xv)
                            )
                    return carry

                lax.fori_loop(0, SG // 16, foldq, 0)

            @pl.when(nsg2 > 0)
            def _():
                fire(0, 0, sem0)

            def apply_pair(gi, carry):
                fire(2 * gi + 1, 1, sem1)
                drain(0, sem0)
                fold(2 * gi, 0)

                @pl.when(gi + 1 < nsg2)
                def _():
                    fire(2 * gi + 2, 0, sem0)

                drain(1, sem1)
                fold(2 * gi + 1, 1)
                return carry

            lax.fori_loop(0, nsg2, apply_pair, 0)
            return carry

        lax.fori_loop(0, nchunk, chunk_body, 0)

        obase = (g_half * out_rows + lo) * 16
        for f in range(NF):
            pltpu.sync_copy(
                aggs[f].at[pl.ds(0, rows_per_r * 16)],
                out_hbm.at[f, pl.ds(obase, rows_per_r * 16)],
            )

    return k(x, src, dst)


def _tc_body(*refs):
    a0s = refs[:NF]
    a1s = refs[NF:2 * NF]
    x_ref, wl_ref, wr_ref, b_ref, o_ref = refs[2 * NF:]
    a = jnp.maximum(
        jnp.concatenate([ref[...] for ref in a0s], axis=1),
        jnp.concatenate([ref[...] for ref in a1s], axis=1),
    )
    a = jnp.where(jnp.isfinite(a), a, 0.0)
    y = jnp.dot(a, wl_ref[...], preferred_element_type=jnp.float32)
    y = y + jnp.dot(x_ref[...], wr_ref[...], preferred_element_type=jnp.float32)
    y = y + b_ref[...]
    m = jnp.max(y, axis=1, keepdims=True)
    z = y - m
    lse = jnp.log(jnp.sum(jnp.exp(z), axis=1, keepdims=True))
    o_ref[...] = z - lse


def _tc_final(a0s, a1s, x, w_l, w_r, b2):
    n, d = x.shape
    c = w_l.shape[1]
    blk = 1000
    grid = n // blk
    bank_spec = pl.BlockSpec((blk, 16), lambda i: (i, 0))
    return pl.pallas_call(
        _tc_body,
        grid=(grid,),
        in_specs=(
            [bank_spec] * (2 * NF)
            + [
                pl.BlockSpec((blk, d), lambda i: (i, 0)),
                pl.BlockSpec((d, c), lambda i: (0, 0)),
                pl.BlockSpec((d, c), lambda i: (0, 0)),
                pl.BlockSpec((1, c), lambda i: (0, 0)),
            ]
        ),
        out_specs=pl.BlockSpec((blk, c), lambda i: (i, 0)),
        out_shape=jax.ShapeDtypeStruct((n, c), jnp.float32),
    )(*a0s, *a1s, x, w_l, w_r, b2)


def kernel(x, edge_index, W_l, W_r, b):
    n, d = x.shape
    e = edge_index.shape[1]
    c = W_l.shape[1]
    rows_per_r = -(-n // (NR * 8)) * 8
    out_rows = NR * rows_per_r

    src = edge_index[0]
    dst = edge_index[1]
    pad = (-e) % (NG_EDGE * CHUNK)
    if pad:
        src = jnp.concatenate([src, jnp.zeros((pad,), jnp.int32)])
        dst = jnp.concatenate([dst, jnp.full((pad,), out_rows, jnp.int32)])

    parts = _sc_segment_max(x, src, dst, rows_per_r)
    banks = parts.reshape(NF, NG_EDGE * out_rows, 16)
    a0s = [banks[f, :n] for f in range(NF)]
    a1s = [banks[f, out_rows:out_rows + n] for f in range(NF)]
    return _tc_final(a0s, a1s, x, W_l, W_r, b.reshape(1, c))


# speedup vs baseline: 4.7663x; 4.7538x over previous
import functools

import jax
import jax.numpy as jnp
from jax import lax
from jax.experimental import pallas as pl
from jax.experimental.pallas import tpu as pltpu
from jax.experimental.pallas import tpu_sc as plsc

NC = 2
NS = 16
NW = NC * NS
NG_EDGE = 2
NR = NW // NG_EDGE
CHUNK = 2000
NGRP = CHUNK // 16
NWIN = 8
NF = 8
SG = 16

_GDN = lax.GatherDimensionNumbers(
    offset_dims=(), collapsed_slice_dims=(0,), start_index_map=(0,)
)


def _vgather(v, idx):
    return lax.gather(
        v, idx[:, None], _GDN, (1,),
        mode=lax.GatherScatterMode.PROMISE_IN_BOUNDS,
    )


def _sc_segment_max(x, src, dst, rows_per_r):
    n, d = x.shape
    e = src.shape[0]
    eh = e // NG_EDGE
    nchunk = eh // CHUNK
    out_rows = NR * rows_per_r

    mesh = plsc.VectorSubcoreMesh(
        core_axis_name="c", subcore_axis_name="s", num_cores=NC, num_subcores=NS
    )

    @functools.partial(
        pl.kernel,
        out_type=jax.ShapeDtypeStruct((NF, NG_EDGE * out_rows * 16), jnp.float32),
        mesh=mesh,
        scratch_types=[
            pltpu.VMEM((2 * (CHUNK + 48),), jnp.int32),
            pltpu.VMEM((2 * (CHUNK + 48),), jnp.int32),
            pltpu.VMEM((16 * NWIN,), jnp.int32),
            pltpu.VMEM((16 * NWIN,), jnp.int32),
            pltpu.VMEM((CHUNK + 128,), jnp.int32),
            pltpu.VMEM((CHUNK + 128,), jnp.int32),
            pltpu.VMEM((2, SG, d), jnp.float32),
        ] + [
            pltpu.VMEM(((rows_per_r + 1) * 16,), jnp.float32) for _ in range(NF)
        ] + [
            pltpu.SemaphoreType.DMA,
            pltpu.SemaphoreType.DMA,
            pltpu.SemaphoreType.DMA,
            pltpu.SemaphoreType.DMA,
        ],
        compiler_params=pltpu.CompilerParams(needs_layout_passes=False),
    )
    def k(x_hbm, src_hbm, dst_hbm, out_hbm, dstc, srcc, cntb, offb,
          idxb, ldb, rows, *rest):
        aggs = rest[:NF]
        sem0, sem1, semd, sems = rest[NF], rest[NF + 1], rest[NF + 2], rest[NF + 3]
        c = lax.axis_index("c")
        s = lax.axis_index("s")
        wid = s * NC + c
        g_half = wid % NG_EDGE
        r = wid // NG_EDGE
        lo = r * rows_per_r
        lane = lax.iota(jnp.int32, 16)

        neg = jnp.full((16,), -jnp.inf, jnp.float32)
        zeros = jnp.zeros((16,), jnp.int32)
        trash = jnp.full((16,), rows_per_r, jnp.int32)
        big = jnp.full((16,), out_rows, jnp.int32)
        lo_v = jnp.full((16,), lo, jnp.int32)
        hi_v = jnp.full((16,), lo + rows_per_r, jnp.int32)

        def init_row(rr, carry):
            for f in range(NF):
                aggs[f][pl.ds(rr * 16, 16)] = neg
            return carry

        lax.fori_loop(0, rows_per_r + 1, init_row, 0)

        def init_buf(i, carry):
            idxb[pl.ds(i * 16, 16)] = zeros
            ldb[pl.ds(i * 16, 16)] = trash
            return carry

        lax.fori_loop(0, (CHUNK + 128) // 16, init_buf, 0)

        for bb in range(2):
            for i in range(3):
                dstc[pl.ds(bb * (CHUNK + 48) + CHUNK + i * 16, 16)] = big
                srcc[pl.ds(bb * (CHUNK + 48) + CHUNK + i * 16, 16)] = zeros

        ebase = g_half * eh

        def fire_chunk(ci, bb):
            base = ebase + ci * CHUNK
            pltpu.async_copy(
                dst_hbm.at[pl.ds(base, CHUNK)],
                dstc.at[pl.ds(bb * (CHUNK + 48), CHUNK)], semd,
            )
            pltpu.async_copy(
                src_hbm.at[pl.ds(base, CHUNK)],
                srcc.at[pl.ds(bb * (CHUNK + 48), CHUNK)], sems,
            )

        def drain_chunk(bb):
            pltpu.make_async_copy(
                dst_hbm.at[pl.ds(0, CHUNK)],
                dstc.at[pl.ds(bb * (CHUNK + 48), CHUNK)], semd,
            ).wait()
            pltpu.make_async_copy(
                src_hbm.at[pl.ds(0, CHUNK)],
                srcc.at[pl.ds(bb * (CHUNK + 48), CHUNK)], sems,
            ).wait()

        fire_chunk(0, 0)

        def chunk_body(ci, carry):
            cb = ci % 2
            cbase = cb * (CHUNK + 48)
            drain_chunk(cb)

            @pl.when((cb == 0) & (ci + 1 < nchunk))
            def _():
                fire_chunk(ci + 1, 1)

            @pl.when((cb == 1) & (ci + 1 < nchunk))
            def _():
                fire_chunk(ci + 1, 0)

            def cnt_win(w, carry):
                b = w * 256
                acc = zeros
                for u in range(16):
                    dv = dstc[pl.ds(cbase + b + u * 16, 16)]
                    m = (dv >= lo_v) & (dv < hi_v)
                    cv = plsc.all_reduce_population_count(m)
                    acc = jnp.where(lane == u, cv, acc)
                cntb[pl.ds(w * 16, 16)] = acc
                return carry

            lax.fori_loop(0, NWIN, cnt_win, 0)

            off = jnp.zeros((16,), jnp.int32)
            l15 = jnp.full((16,), 15, jnp.int32)
            for w in range(NWIN):
                cv = cntb[pl.ds(w * 16, 16)]
                cs = jnp.cumsum(cv)
                offb[pl.ds(w * 16, 16)] = off + (cs - cv)
                off = off + _vgather(cs, l15)
            cnt = jnp.max(off)

            def scat_win(w, carry):
                b = w * 256
                ov = offb[pl.ds(w * 16, 16)]
                for u in range(16):
                    dv = dstc[pl.ds(cbase + b + u * 16, 16)]
                    sv = srcc[pl.ds(cbase + b + u * 16, 16)]
                    m = (dv >= lo_v) & (dv < hi_v)
                    mi = m.astype(jnp.int32)
                    cs = jnp.cumsum(mi)
                    osp = _vgather(ov, jnp.full((16,), u, jnp.int32))
                    addr = cs + osp - 1
                    plsc.store_scatter(idxb, [addr], sv, mask=m)
                    plsc.store_scatter(ldb, [addr], dv - lo_v, mask=m)
                return carry

            lax.fori_loop(0, NWIN, scat_win, 0)

            nsg2 = (cnt + 2 * SG - 1) // (2 * SG)

            def fire(sg, buf, sem):
                pltpu.async_copy(
                    x_hbm.at[idxb.at[pl.ds(sg * SG, SG)]], rows.at[buf], sem
                )

            def drain(buf, sem):
                pltpu.make_async_copy(
                    x_hbm.at[idxb.at[pl.ds(0, SG)]], rows.at[buf], sem
                ).wait()

            def fold(sg, buf):
                def foldq(q, carry):
                    ldv = ldb[pl.ds(sg * SG + q * 16, 16)]
                    base = ldv * 16
                    addrs = [
                        _vgather(base, jnp.full((16,), j, jnp.int32)) + lane
                        for j in range(16)
                    ]
                    for j in range(16):
                        for f in range(NF):
                            cur = plsc.load_gather(aggs[f], [addrs[j]])
                            xv = rows[buf, q * 16 + j, pl.ds(f * 16, 16)]
                            plsc.store_scatter(
                                aggs[f], [addrs[j]], jnp.maximum(cur, xv)
                            )
                    return carry

                lax.fori_loop(0, SG // 16, foldq, 0)

            @pl.when(nsg2 > 0)
            def _():
                fire(0, 0, sem0)

            def apply_pair(gi, carry):
                fire(2 * gi + 1, 1, sem1)
                drain(0, sem0)
                fold(2 * gi, 0)

                @pl.when(gi + 1 < nsg2)
                def _():
                    fire(2 * gi + 2, 0, sem0)

                drain(1, sem1)
                fold(2 * gi + 1, 1)
                return carry

            lax.fori_loop(0, nsg2, apply_pair, 0)
            return carry

        lax.fori_loop(0, nchunk, chunk_body, 0)

        obase = (g_half * out_rows + lo) * 16
        for f in range(NF):
            pltpu.sync_copy(
                aggs[f].at[pl.ds(0, rows_per_r * 16)],
                out_hbm.at[f, pl.ds(obase, rows_per_r * 16)],
            )

    return k(x, src, dst)


def _tc_body(*refs):
    a0s = refs[:NF]
    a1s = refs[NF:2 * NF]
    x_ref, wl_ref, wr_ref, b_ref, o_ref = refs[2 * NF:]
    a = jnp.maximum(
        jnp.concatenate([ref[...] for ref in a0s], axis=1),
        jnp.concatenate([ref[...] for ref in a1s], axis=1),
    )
    a = jnp.where(jnp.isfinite(a), a, 0.0)
    y = jnp.dot(a, wl_ref[...], preferred_element_type=jnp.float32)
    y = y + jnp.dot(x_ref[...], wr_ref[...], preferred_element_type=jnp.float32)
    y = y + b_ref[...]
    m = jnp.max(y, axis=1, keepdims=True)
    z = y - m
    lse = jnp.log(jnp.sum(jnp.exp(z), axis=1, keepdims=True))
    o_ref[...] = z - lse


def _tc_final(a0s, a1s, x, w_l, w_r, b2):
    n, d = x.shape
    c = w_l.shape[1]
    blk = 1000
    grid = n // blk
    bank_spec = pl.BlockSpec((blk, 16), lambda i: (i, 0))
    return pl.pallas_call(
        _tc_body,
        grid=(grid,),
        in_specs=(
            [bank_spec] * (2 * NF)
            + [
                pl.BlockSpec((blk, d), lambda i: (i, 0)),
                pl.BlockSpec((d, c), lambda i: (0, 0)),
                pl.BlockSpec((d, c), lambda i: (0, 0)),
                pl.BlockSpec((1, c), lambda i: (0, 0)),
            ]
        ),
        out_specs=pl.BlockSpec((blk, c), lambda i: (i, 0)),
        out_shape=jax.ShapeDtypeStruct((n, c), jnp.float32),
    )(*a0s, *a1s, x, w_l, w_r, b2)


def kernel(x, edge_index, W_l, W_r, b):
    n, d = x.shape
    e = edge_index.shape[1]
    c = W_l.shape[1]
    rows_per_r = -(-n // (NR * 8)) * 8
    out_rows = NR * rows_per_r

    src = edge_index[0]
    dst = edge_index[1]
    pad = (-e) % (NG_EDGE * CHUNK)
    if pad:
        src = jnp.concatenate([src, jnp.zeros((pad,), jnp.int32)])
        dst = jnp.concatenate([dst, jnp.full((pad,), out_rows, jnp.int32)])

    parts = _sc_segment_max(x, src, dst, rows_per_r)
    banks = parts.reshape(NF, NG_EDGE * out_rows, 16)
    a0s = [banks[f, :n] for f in range(NF)]
    a1s = [banks[f, out_rows:out_rows + n] for f in range(NF)]
    return _tc_final(a0s, a1s, x, W_l, W_r, b.reshape(1, c))
